# Initial kernel scaffold; baseline (speedup 1.0000x reference)
import jax
import jax.numpy as jnp
from jax.experimental import pallas as pl


def kernel(x, adj, W_emb, W_fc, bias, prelu_a):
    raise NotImplementedError("write your pallas kernel here")


# speedup vs baseline: 1.2941x; 1.2941x over previous
import jax
import jax.numpy as jnp
from jax.experimental import pallas as pl
from jax.experimental.pallas import tpu as pltpu

_BLK1 = 200
_BLK2 = 400


def _stage1_body(x_ref, we_ref, wf_ref, b_ref, a_ref, adj_ref,
                 lh_ref, adjb_ref, h_scr):
    i = pl.program_id(0)

    @pl.when(i == 0)
    def _():
        xe = jax.lax.dot_general(
            x_ref[...], we_ref[...], (((1,), (0,)), ((), ())),
            preferred_element_type=jnp.float32)
        h = jax.lax.dot_general(
            xe, wf_ref[...], (((1,), (1,)), ((), ())),
            preferred_element_type=jnp.float32)
        h_scr[...] = h.astype(jnp.bfloat16)

    adjb = adj_ref[...].astype(jnp.bfloat16)
    adjb_ref[...] = adjb
    t = jax.lax.dot_general(adjb, h_scr[...], (((1,), (0,)), ((), ())),
                            preferred_element_type=jnp.float32)
    t = t + b_ref[...]
    a = a_ref[0, 0]
    lh_ref[...] = jnp.where(t >= 0.0, t, a * t)


def _stage2_body(lh_ref, adjb_ref, out_ref, g0, g1):
    s = pl.program_id(0)
    i = pl.program_id(1)
    adjb = adjb_ref[...]
    row = i * _BLK2

    def prop(src):
        return jax.lax.dot_general(adjb, src, (((1,), (0,)), ((), ())),
                                   preferred_element_type=jnp.float32)

    @pl.when(s == 0)
    def _():
        g = prop(lh_ref[...].astype(jnp.bfloat16))
        g0[pl.ds(row, _BLK2), :] = g.astype(jnp.bfloat16)

    @pl.when((s == 1) | (s == 3))
    def _():
        g1[pl.ds(row, _BLK2), :] = prop(g0[...]).astype(jnp.bfloat16)

    @pl.when(s == 2)
    def _():
        g0[pl.ds(row, _BLK2), :] = prop(g1[...]).astype(jnp.bfloat16)

    @pl.when(s == 4)
    def _():
        hh = lh_ref[pl.ds(row, _BLK2), :] + prop(g1[...])
        nrm = jnp.sqrt(jnp.sum(hh * hh, axis=-1, keepdims=True))
        out_ref[...] = hh / jnp.maximum(nrm, 1e-12)


def kernel(x, adj, W_emb, W_fc, bias, prelu_a):
    n = x.shape[1]
    d = x.shape[2]
    x2 = x.reshape(n, d)
    adj2 = adj.reshape(n, n)
    bias2 = bias.reshape(1, d)
    a2 = jnp.reshape(prelu_a, (1, 1)).astype(jnp.float32)

    lh, adjb = pl.pallas_call(
        _stage1_body,
        grid=(n // _BLK1,),
        in_specs=[
            pl.BlockSpec((n, d), lambda i: (0, 0)),
            pl.BlockSpec((d, d), lambda i: (0, 0)),
            pl.BlockSpec((d, d), lambda i: (0, 0)),
            pl.BlockSpec((1, d), lambda i: (0, 0)),
            pl.BlockSpec((1, 1), lambda i: (0, 0)),
            pl.BlockSpec((_BLK1, n), lambda i: (i, 0)),
        ],
        out_specs=[
            pl.BlockSpec((_BLK1, d), lambda i: (i, 0)),
            pl.BlockSpec((_BLK1, n), lambda i: (i, 0)),
        ],
        out_shape=[
            jax.ShapeDtypeStruct((n, d), jnp.float32),
            jax.ShapeDtypeStruct((n, n), jnp.bfloat16),
        ],
        scratch_shapes=[pltpu.VMEM((n, d), jnp.bfloat16)],
        compiler_params=pltpu.CompilerParams(
            dimension_semantics=("arbitrary",)),
    )(x2, W_emb, W_fc, bias2, a2, adj2)

    out = pl.pallas_call(
        _stage2_body,
        grid=(5, n // _BLK2),
        in_specs=[
            pl.BlockSpec((n, d), lambda s, i: (0, 0)),
            pl.BlockSpec((_BLK2, n), lambda s, i: (i, 0)),
        ],
        out_specs=pl.BlockSpec((_BLK2, d), lambda s, i: (i, 0)),
        out_shape=jax.ShapeDtypeStruct((n, d), jnp.float32),
        scratch_shapes=[
            pltpu.VMEM((n, d), jnp.bfloat16),
            pltpu.VMEM((n, d), jnp.bfloat16),
        ],
        compiler_params=pltpu.CompilerParams(
            dimension_semantics=("arbitrary", "arbitrary")),
    )(lh, adjb)

    return out[None, :, :]


# speedup vs baseline: 1.4992x; 1.1585x over previous
import functools

import jax
import jax.numpy as jnp
from jax.experimental import pallas as pl
from jax.experimental.pallas import tpu as pltpu

_BLK1 = 200
_BLK2 = 1000


def _stage1_body(x_ref, we_ref, wf_ref, b_ref, a_ref, adj_ref,
                 lh_ref, adjb_ref, h_scr):
    i = pl.program_id(0)

    @pl.when(i == 0)
    def _():
        xe = jax.lax.dot_general(
            x_ref[...], we_ref[...], (((1,), (0,)), ((), ())),
            preferred_element_type=jnp.float32)
        h = jax.lax.dot_general(
            xe, wf_ref[...], (((1,), (1,)), ((), ())),
            preferred_element_type=jnp.float32)
        h_scr[...] = h.astype(jnp.bfloat16)

    adjb_ref[...] = adj_ref[...].astype(jnp.bfloat16)
    t = jax.lax.dot_general(adjb_ref[...], h_scr[...], (((1,), (0,)), ((), ())),
                            preferred_element_type=jnp.float32)
    t = t + b_ref[...]
    a = a_ref[0, 0]
    lh_ref[...] = jnp.where(t >= 0.0, t, a * t).astype(jnp.bfloat16)


def _stage2_body(blk2, lh_ref, adjb_ref, out_ref, g0, g1):
    s = pl.program_id(0)
    i = pl.program_id(1)
    row = i * blk2

    def prop(src):
        return jax.lax.dot_general(adjb_ref[...], src, (((1,), (0,)), ((), ())),
                                   preferred_element_type=jnp.float32)

    @pl.when(s == 0)
    def _():
        g0[pl.ds(row, blk2), :] = prop(lh_ref[...]).astype(jnp.bfloat16)

    @pl.when((s == 1) | (s == 3))
    def _():
        g1[pl.ds(row, blk2), :] = prop(g0[...]).astype(jnp.bfloat16)

    @pl.when(s == 2)
    def _():
        g0[pl.ds(row, blk2), :] = prop(g1[...]).astype(jnp.bfloat16)

    @pl.when(s == 4)
    def _():
        hh = lh_ref[pl.ds(row, blk2), :].astype(jnp.float32) + prop(g1[...])
        nrm = jnp.sqrt(jnp.sum(hh * hh, axis=-1, keepdims=True))
        out_ref[...] = hh / jnp.maximum(nrm, 1e-12)


def kernel(x, adj, W_emb, W_fc, bias, prelu_a):
    n = x.shape[1]
    d = x.shape[2]
    x2 = x.reshape(n, d)
    adj2 = adj.reshape(n, n)
    bias2 = bias.reshape(1, d)
    a2 = jnp.reshape(prelu_a, (1, 1)).astype(jnp.float32)

    lh, adjb = pl.pallas_call(
        _stage1_body,
        grid=(n // _BLK1,),
        in_specs=[
            pl.BlockSpec((n, d), lambda i: (0, 0)),
            pl.BlockSpec((d, d), lambda i: (0, 0)),
            pl.BlockSpec((d, d), lambda i: (0, 0)),
            pl.BlockSpec((1, d), lambda i: (0, 0)),
            pl.BlockSpec((1, 1), lambda i: (0, 0)),
            pl.BlockSpec((_BLK1, n), lambda i: (i, 0)),
        ],
        out_specs=[
            pl.BlockSpec((_BLK1, d), lambda i: (i, 0)),
            pl.BlockSpec((_BLK1, n), lambda i: (i, 0)),
        ],
        out_shape=[
            jax.ShapeDtypeStruct((n, d), jnp.bfloat16),
            jax.ShapeDtypeStruct((n, n), jnp.bfloat16),
        ],
        scratch_shapes=[pltpu.VMEM((n, d), jnp.bfloat16)],
        compiler_params=pltpu.CompilerParams(
            dimension_semantics=("arbitrary",)),
    )(x2, W_emb, W_fc, bias2, a2, adj2)

    blk2 = min(_BLK2, n)
    out = pl.pallas_call(
        functools.partial(_stage2_body, blk2),
        grid=(5, n // blk2),
        in_specs=[
            pl.BlockSpec((n, d), lambda s, i: (0, 0)),
            pl.BlockSpec((blk2, n), lambda s, i: (i, 0)),
        ],
        out_specs=pl.BlockSpec((blk2, d), lambda s, i: (i, 0)),
        out_shape=jax.ShapeDtypeStruct((n, d), jnp.float32),
        scratch_shapes=[
            pltpu.VMEM((n, d), jnp.bfloat16),
            pltpu.VMEM((n, d), jnp.bfloat16),
        ],
        compiler_params=pltpu.CompilerParams(
            dimension_semantics=("arbitrary", "arbitrary")),
    )(lh, adjb)

    return out[None, :, :]


# speedup vs baseline: 1.5184x; 1.0128x over previous
import functools

import jax
import jax.numpy as jnp
from jax.experimental import pallas as pl
from jax.experimental.pallas import tpu as pltpu

_BLK1 = 400
_BLK2 = 1000


def _stage1_body(x_ref, we_ref, wf_ref, b_ref, a_ref, adj_ref,
                 lh_ref, adjb_ref, h_scr):
    i = pl.program_id(0)

    @pl.when(i == 0)
    def _():
        xe = jax.lax.dot_general(
            x_ref[...], we_ref[...], (((1,), (0,)), ((), ())),
            preferred_element_type=jnp.float32)
        h = jax.lax.dot_general(
            xe, wf_ref[...], (((1,), (1,)), ((), ())),
            preferred_element_type=jnp.float32)
        h_scr[...] = h.astype(jnp.bfloat16)

    adjb_ref[...] = adj_ref[...].astype(jnp.bfloat16)
    t = jax.lax.dot_general(adjb_ref[...], h_scr[...], (((1,), (0,)), ((), ())),
                            preferred_element_type=jnp.float32)
    t = t + b_ref[...]
    a = a_ref[0, 0]
    lh_ref[...] = jnp.where(t >= 0.0, t, a * t).astype(jnp.bfloat16)


def _stage2_body(blk2, lh_ref, adjb_ref, out_ref, g0, g1):
    s = pl.program_id(0)
    i = pl.program_id(1)
    row = i * blk2

    def prop(src):
        return jax.lax.dot_general(adjb_ref[...], src, (((1,), (0,)), ((), ())),
                                   preferred_element_type=jnp.float32)

    @pl.when(s == 0)
    def _():
        g0[pl.ds(row, blk2), :] = prop(lh_ref[...]).astype(jnp.bfloat16)

    @pl.when((s == 1) | (s == 3))
    def _():
        g1[pl.ds(row, blk2), :] = prop(g0[...]).astype(jnp.bfloat16)

    @pl.when(s == 2)
    def _():
        g0[pl.ds(row, blk2), :] = prop(g1[...]).astype(jnp.bfloat16)

    @pl.when(s == 4)
    def _():
        hh = lh_ref[pl.ds(row, blk2), :].astype(jnp.float32) + prop(g1[...])
        nrm = jnp.sqrt(jnp.sum(hh * hh, axis=-1, keepdims=True))
        out_ref[...] = hh / jnp.maximum(nrm, 1e-12)


def kernel(x, adj, W_emb, W_fc, bias, prelu_a):
    n = x.shape[1]
    d = x.shape[2]
    x2 = x.reshape(n, d)
    adj2 = adj.reshape(n, n)
    bias2 = bias.reshape(1, d)
    a2 = jnp.reshape(prelu_a, (1, 1)).astype(jnp.float32)

    lh, adjb = pl.pallas_call(
        _stage1_body,
        grid=(n // _BLK1,),
        in_specs=[
            pl.BlockSpec((n, d), lambda i: (0, 0)),
            pl.BlockSpec((d, d), lambda i: (0, 0)),
            pl.BlockSpec((d, d), lambda i: (0, 0)),
            pl.BlockSpec((1, d), lambda i: (0, 0)),
            pl.BlockSpec((1, 1), lambda i: (0, 0)),
            pl.BlockSpec((_BLK1, n), lambda i: (i, 0)),
        ],
        out_specs=[
            pl.BlockSpec((_BLK1, d), lambda i: (i, 0)),
            pl.BlockSpec((_BLK1, n), lambda i: (i, 0)),
        ],
        out_shape=[
            jax.ShapeDtypeStruct((n, d), jnp.bfloat16),
            jax.ShapeDtypeStruct((n, n), jnp.bfloat16),
        ],
        scratch_shapes=[pltpu.VMEM((n, d), jnp.bfloat16)],
        compiler_params=pltpu.CompilerParams(
            dimension_semantics=("arbitrary",)),
    )(x2, W_emb, W_fc, bias2, a2, adj2)

    blk2 = min(_BLK2, n)
    out = pl.pallas_call(
        functools.partial(_stage2_body, blk2),
        grid=(5, n // blk2),
        in_specs=[
            pl.BlockSpec((n, d), lambda s, i: (0, 0)),
            pl.BlockSpec((blk2, n), lambda s, i: (i, 0)),
        ],
        out_specs=pl.BlockSpec((blk2, d), lambda s, i: (i, 0)),
        out_shape=jax.ShapeDtypeStruct((n, d), jnp.float32),
        scratch_shapes=[
            pltpu.VMEM((n, d), jnp.bfloat16),
            pltpu.VMEM((n, d), jnp.bfloat16),
        ],
        compiler_params=pltpu.CompilerParams(
            dimension_semantics=("arbitrary", "arbitrary")),
    )(lh, adjb)

    return out[None, :, :]


# speedup vs baseline: 1.5376x; 1.0127x over previous
import functools

import jax
import jax.numpy as jnp
from jax.experimental import pallas as pl
from jax.experimental.pallas import tpu as pltpu

_BLK1 = 400
_BLK2 = 1000


def _stage1_body(x_ref, we_ref, wf_ref, b_ref, a_ref, adj_ref,
                 lh_ref, adjb_ref, h_scr):
    i = pl.program_id(0)

    @pl.when(i == 0)
    def _():
        xe = jax.lax.dot_general(
            x_ref[...], we_ref[...], (((1,), (0,)), ((), ())),
            preferred_element_type=jnp.float32)
        h = jax.lax.dot_general(
            xe, wf_ref[...], (((1,), (1,)), ((), ())),
            preferred_element_type=jnp.float32)
        h_scr[...] = h.astype(jnp.bfloat16)

    adjb_ref[...] = adj_ref[...].astype(jnp.bfloat16)
    t = jax.lax.dot_general(adjb_ref[...], h_scr[...], (((1,), (0,)), ((), ())),
                            preferred_element_type=jnp.float32)
    t = t + b_ref[...]
    a = a_ref[0, 0]
    lh_ref[...] = jnp.where(t >= 0.0, t, a * t).astype(jnp.bfloat16)


def _stage2_body(blk2, lh_ref, adjb_ref, out_ref, g0, g1, gb):
    s = pl.program_id(0)
    i = pl.program_id(1)
    row = i * blk2

    @pl.when(((s == 1) | (s == 3)) & (i == 0))
    def _():
        gb[...] = g0[...].astype(jnp.bfloat16)

    @pl.when(((s == 2) | (s == 4)) & (i == 0))
    def _():
        gb[...] = g1[...].astype(jnp.bfloat16)

    def prop(src):
        return jax.lax.dot_general(adjb_ref[...], src, (((1,), (0,)), ((), ())),
                                   preferred_element_type=jnp.float32)

    @pl.when(s == 0)
    def _():
        g0[pl.ds(row, blk2), :] = prop(lh_ref[...])

    @pl.when((s == 1) | (s == 3))
    def _():
        g1[pl.ds(row, blk2), :] = prop(gb[...])

    @pl.when(s == 2)
    def _():
        g0[pl.ds(row, blk2), :] = prop(gb[...])

    @pl.when(s == 4)
    def _():
        hh = lh_ref[pl.ds(row, blk2), :].astype(jnp.float32) + prop(gb[...])
        nrm = jnp.sqrt(jnp.sum(hh * hh, axis=-1, keepdims=True))
        out_ref[...] = hh / jnp.maximum(nrm, 1e-12)


def kernel(x, adj, W_emb, W_fc, bias, prelu_a):
    n = x.shape[1]
    d = x.shape[2]
    x2 = x.reshape(n, d)
    adj2 = adj.reshape(n, n)
    bias2 = bias.reshape(1, d)
    a2 = jnp.reshape(prelu_a, (1, 1)).astype(jnp.float32)

    lh, adjb = pl.pallas_call(
        _stage1_body,
        grid=(n // _BLK1,),
        in_specs=[
            pl.BlockSpec((n, d), lambda i: (0, 0)),
            pl.BlockSpec((d, d), lambda i: (0, 0)),
            pl.BlockSpec((d, d), lambda i: (0, 0)),
            pl.BlockSpec((1, d), lambda i: (0, 0)),
            pl.BlockSpec((1, 1), lambda i: (0, 0)),
            pl.BlockSpec((_BLK1, n), lambda i: (i, 0)),
        ],
        out_specs=[
            pl.BlockSpec((_BLK1, d), lambda i: (i, 0)),
            pl.BlockSpec((_BLK1, n), lambda i: (i, 0)),
        ],
        out_shape=[
            jax.ShapeDtypeStruct((n, d), jnp.bfloat16),
            jax.ShapeDtypeStruct((n, n), jnp.bfloat16),
        ],
        scratch_shapes=[pltpu.VMEM((n, d), jnp.bfloat16)],
        compiler_params=pltpu.CompilerParams(
            dimension_semantics=("arbitrary",)),
    )(x2, W_emb, W_fc, bias2, a2, adj2)

    blk2 = min(_BLK2, n)
    out = pl.pallas_call(
        functools.partial(_stage2_body, blk2),
        grid=(5, n // blk2),
        in_specs=[
            pl.BlockSpec((n, d), lambda s, i: (0, 0)),
            pl.BlockSpec((blk2, n), lambda s, i: (i, 0)),
        ],
        out_specs=pl.BlockSpec(
            (blk2, d), lambda s, i: (jnp.where(s == 4, i, 0), 0)),
        out_shape=jax.ShapeDtypeStruct((n, d), jnp.float32),
        scratch_shapes=[
            pltpu.VMEM((n, d), jnp.float32),
            pltpu.VMEM((n, d), jnp.float32),
            pltpu.VMEM((n, d), jnp.bfloat16),
        ],
        compiler_params=pltpu.CompilerParams(
            dimension_semantics=("arbitrary", "arbitrary")),
    )(lh, adjb)

    return out[None, :, :]
